# baseline (speedup 1.0000x reference)
import functools

import jax
import jax.numpy as jnp
from jax import lax
from jax.experimental import pallas as pl
from jax.experimental.pallas import tpu as pltpu
from jax.experimental.pallas import tpu_sc as plsc

_DEFAULT = jax.lax.Precision.DEFAULT

FEATURE_DIM = 1024
HIDDEN = 512
CODEBOOK_DIM = 512
CODEBOOK_SIZE = 256

TM1 = 512
TM2 = 2048


def _scores_body(x_ref, w1_ref, b1_ref, w2_ref, b2_ref, cb_ref, sq_ref):
    h = jnp.maximum(
        jnp.dot(x_ref[...], w1_ref[...], preferred_element_type=jnp.float32,
                precision=_DEFAULT) + b1_ref[...], 0.0)
    e = jnp.dot(h, w2_ref[...], preferred_element_type=jnp.float32,
                precision=_DEFAULT) + b2_ref[...]
    cb = cb_ref[...]
    scores = jax.lax.dot_general(
        e, cb, (((1,), (1,)), ((), ())),
        preferred_element_type=jnp.float32, precision=_DEFAULT)
    enorm = jnp.sum(e * e, axis=1, keepdims=True)
    cnorm = jnp.sum(cb * cb, axis=1)[None, :]
    sq_ref[...] = enorm + cnorm - 2.0 * scores


def _argmin_body(sq_ref, idx_ref, loss_ref):
    sq = sq_ref[...]
    dist = jnp.sqrt(jnp.maximum(sq, 0.0))
    iota = jax.lax.broadcasted_iota(jnp.int32, (TM2, CODEBOOK_SIZE), 1)
    dmin = jnp.min(dist, axis=1, keepdims=True)
    idx = jnp.min(jnp.where(dist == dmin, iota, CODEBOOK_SIZE),
                  axis=1).astype(jnp.int32)
    idx_ref[0, 0, :] = idx
    onehot = (idx[:, None] == iota).astype(jnp.float32)
    sel = jnp.maximum(jnp.sum(onehot * sq, axis=1), 0.0)
    part = jnp.sum(sel, keepdims=True)[None, :]

    @pl.when(pl.program_id(0) == 0)
    def _():
        loss_ref[...] = jnp.zeros_like(part)

    loss_ref[...] += part


def _make_gather(n):
    info = plsc.get_sparse_core_info()
    nw = info.num_cores * info.num_subcores
    b_per_w = n // nw
    chunk = 64
    nchunks = b_per_w // chunk
    mesh = plsc.VectorSubcoreMesh(core_axis_name="c", subcore_axis_name="s")

    @functools.partial(
        pl.kernel, mesh=mesh,
        out_type=jax.ShapeDtypeStruct((n, CODEBOOK_DIM), jnp.float32),
        scratch_types=[
            pltpu.VMEM((b_per_w,), jnp.int32),
            pltpu.VMEM((2, chunk, CODEBOOK_DIM), jnp.float32),
            pltpu.SemaphoreType.DMA,
            pltpu.SemaphoreType.DMA,
        ],
    )
    def gather(table_hbm, idx_hbm, out_hbm, idx_v, rows_v, sem0, sem1):
        wid = lax.axis_index("s") * info.num_cores + lax.axis_index("c")
        base = wid * b_per_w
        pltpu.sync_copy(idx_hbm.at[pl.ds(base, b_per_w)], idx_v)
        sems = (sem0, sem1)

        def fire(c, slot):
            pltpu.async_copy(
                table_hbm.at[idx_v.at[pl.ds(c * chunk, chunk)]],
                rows_v.at[slot], sems[slot])

        fire(0, 0)
        for c in range(nchunks):
            slot = c % 2
            if c + 1 < nchunks:
                fire(c + 1, 1 - slot)
            pltpu.make_async_copy(
                table_hbm.at[idx_v.at[pl.ds(c * chunk, chunk)]],
                rows_v.at[slot], sems[slot]).wait()
            pltpu.sync_copy(rows_v.at[slot],
                            out_hbm.at[pl.ds(base + c * chunk, chunk)])

    return gather


TM4 = 1024


def _tc_gather_body(idx_ref, cb_ref, q_ref):
    idx = idx_ref[0, 0, :]
    onehot = (idx[:, None] == jax.lax.broadcasted_iota(
        jnp.int32, (TM4, CODEBOOK_SIZE), 1)).astype(jnp.float32)
    q_ref[...] = jnp.dot(onehot, cb_ref[...],
                         preferred_element_type=jnp.float32,
                         precision=jax.lax.Precision.HIGHEST)


def _tc_gather(idx_flat, codebook):
    m = idx_flat.shape[0]
    g = m // TM4
    return pl.pallas_call(
        _tc_gather_body,
        grid=(g,),
        in_specs=[
            pl.BlockSpec((1, 1, TM4), lambda i: (i, 0, 0)),
            pl.BlockSpec((CODEBOOK_SIZE, CODEBOOK_DIM), lambda i: (0, 0)),
        ],
        out_specs=[pl.BlockSpec((TM4, CODEBOOK_DIM), lambda i: (i, 0))],
        out_shape=[jax.ShapeDtypeStruct((m, CODEBOOK_DIM), jnp.float32)],
    )(idx_flat.reshape(g, 1, TM4), codebook)[0]


N_SC = 4096


def _encode_chunk(xc, W1, b1r, W2, b2r, codebook):
    nc, Dx = xc.shape
    sq = pl.pallas_call(
        _scores_body,
        grid=(nc // TM1,),
        in_specs=[
            pl.BlockSpec((TM1, Dx), lambda i: (i, 0)),
            pl.BlockSpec((Dx, HIDDEN), lambda i: (0, 0)),
            pl.BlockSpec((1, HIDDEN), lambda i: (0, 0)),
            pl.BlockSpec((HIDDEN, CODEBOOK_DIM), lambda i: (0, 0)),
            pl.BlockSpec((1, CODEBOOK_DIM), lambda i: (0, 0)),
            pl.BlockSpec((CODEBOOK_SIZE, CODEBOOK_DIM), lambda i: (0, 0)),
        ],
        out_specs=[pl.BlockSpec((TM1, CODEBOOK_SIZE), lambda i: (i, 0))],
        out_shape=[jax.ShapeDtypeStruct((nc, CODEBOOK_SIZE), jnp.float32)],
    )(xc, W1, b1r, W2, b2r, codebook)[0]

    g2 = nc // TM2
    idx_out, loss_out = pl.pallas_call(
        _argmin_body,
        grid=(g2,),
        in_specs=[pl.BlockSpec((TM2, CODEBOOK_SIZE), lambda i: (i, 0))],
        out_specs=[
            pl.BlockSpec((1, 1, TM2), lambda i: (i, 0, 0)),
            pl.BlockSpec((1, 1), lambda i: (0, 0)),
        ],
        out_shape=[
            jax.ShapeDtypeStruct((g2, 1, TM2), jnp.int32),
            jax.ShapeDtypeStruct((1, 1), jnp.float32),
        ],
    )(sq)
    return idx_out.reshape(nc), loss_out[0, 0]


@jax.jit
def kernel(x, W1, b1, W2, b2, codebook):
    Bx, Tx, Dx = x.shape
    n = Bx * Tx
    xf = x.reshape(n, Dx)
    b1r = b1.reshape(1, HIDDEN)
    b2r = b2.reshape(1, CODEBOOK_DIM)

    idx_flat, loss_sum = _encode_chunk(xf, W1, b1r, W2, b2r, codebook)

    q_sc = _make_gather(N_SC)(codebook, jax.lax.slice(idx_flat, (0,), (N_SC,)))
    q_tc = _tc_gather(jax.lax.slice(idx_flat, (N_SC,), (n,)), codebook)
    quantized = jnp.concatenate([q_sc, q_tc]).reshape(Bx, Tx, CODEBOOK_DIM)

    token_indices = idx_flat.reshape(Bx, Tx)
    denom = jnp.float32(n * CODEBOOK_DIM)
    commitment_loss = loss_sum / denom
    codebook_loss = commitment_loss
    vq_loss = commitment_loss + 0.25 * codebook_loss
    return (token_indices, quantized, commitment_loss, codebook_loss,
            vq_loss)


# speedup vs baseline: 1.0384x; 1.0384x over previous
import functools

import jax
import jax.numpy as jnp
from jax import lax
from jax.experimental import pallas as pl
from jax.experimental.pallas import tpu as pltpu
from jax.experimental.pallas import tpu_sc as plsc

_DEFAULT = jax.lax.Precision.DEFAULT

FEATURE_DIM = 1024
HIDDEN = 512
CODEBOOK_DIM = 512
CODEBOOK_SIZE = 256

TM1 = 512
TM2 = 2048


def _scores_body(x_ref, w1_ref, b1_ref, w2_ref, b2_ref, cb_ref, sq_ref):
    h = jnp.maximum(
        jnp.dot(x_ref[...], w1_ref[...], preferred_element_type=jnp.float32,
                precision=_DEFAULT) + b1_ref[...], 0.0)
    e = jnp.dot(h, w2_ref[...], preferred_element_type=jnp.float32,
                precision=_DEFAULT) + b2_ref[...]
    cb = cb_ref[...]
    scores = jax.lax.dot_general(
        e, cb, (((1,), (1,)), ((), ())),
        preferred_element_type=jnp.float32, precision=_DEFAULT)
    enorm = jnp.sum(e * e, axis=1, keepdims=True)
    cnorm = jnp.sum(cb * cb, axis=1)[None, :]
    sq_ref[...] = enorm + cnorm - 2.0 * scores


def _argmin_body(sq_ref, idx_ref, loss_ref):
    sq = sq_ref[...]
    dist = jnp.sqrt(jnp.maximum(sq, 0.0))
    iota = jax.lax.broadcasted_iota(jnp.int32, (TM2, CODEBOOK_SIZE), 1)
    dmin = jnp.min(dist, axis=1, keepdims=True)
    idx = jnp.min(jnp.where(dist == dmin, iota, CODEBOOK_SIZE),
                  axis=1).astype(jnp.int32)
    idx_ref[0, 0, :] = idx
    onehot = (idx[:, None] == iota).astype(jnp.float32)
    sel = jnp.maximum(jnp.sum(onehot * sq, axis=1), 0.0)
    part = jnp.sum(sel, keepdims=True)[None, :]

    @pl.when(pl.program_id(0) == 0)
    def _():
        loss_ref[...] = jnp.zeros_like(part)

    loss_ref[...] += part


def _make_gather(n):
    info = plsc.get_sparse_core_info()
    nw = info.num_cores * info.num_subcores
    b_per_w = n // nw
    chunk = 64
    nchunks = b_per_w // chunk
    mesh = plsc.VectorSubcoreMesh(core_axis_name="c", subcore_axis_name="s")

    @functools.partial(
        pl.kernel, mesh=mesh,
        out_type=jax.ShapeDtypeStruct((n, CODEBOOK_DIM), jnp.float32),
        scratch_types=[
            pltpu.VMEM((b_per_w,), jnp.int32),
            pltpu.VMEM((2, chunk, CODEBOOK_DIM), jnp.float32),
            pltpu.SemaphoreType.DMA,
            pltpu.SemaphoreType.DMA,
        ],
    )
    def gather(table_hbm, idx_hbm, out_hbm, idx_v, rows_v, sem0, sem1):
        wid = lax.axis_index("s") * info.num_cores + lax.axis_index("c")
        base = wid * b_per_w
        pltpu.sync_copy(idx_hbm.at[pl.ds(base, b_per_w)], idx_v)
        sems = (sem0, sem1)

        def fire(c, slot):
            pltpu.async_copy(
                table_hbm.at[idx_v.at[pl.ds(c * chunk, chunk)]],
                rows_v.at[slot], sems[slot])

        fire(0, 0)
        for c in range(nchunks):
            slot = c % 2
            if c + 1 < nchunks:
                fire(c + 1, 1 - slot)
            pltpu.make_async_copy(
                table_hbm.at[idx_v.at[pl.ds(c * chunk, chunk)]],
                rows_v.at[slot], sems[slot]).wait()
            pltpu.sync_copy(rows_v.at[slot],
                            out_hbm.at[pl.ds(base + c * chunk, chunk)])

    return gather


TM4 = 1024


def _tc_gather_body(idx_ref, cb_ref, q_ref):
    idx = idx_ref[0, 0, :]
    onehot = (idx[:, None] == jax.lax.broadcasted_iota(
        jnp.int32, (TM4, CODEBOOK_SIZE), 1)).astype(jnp.float32)
    cb = cb_ref[...]
    cb_hi = cb.astype(jnp.bfloat16).astype(jnp.float32)
    cb_lo = cb - cb_hi
    q_ref[...] = (
        jnp.dot(onehot, cb_hi, preferred_element_type=jnp.float32,
                precision=_DEFAULT)
        + jnp.dot(onehot, cb_lo, preferred_element_type=jnp.float32,
                  precision=_DEFAULT))


def _tc_gather(idx_flat, codebook):
    m = idx_flat.shape[0]
    g = m // TM4
    return pl.pallas_call(
        _tc_gather_body,
        grid=(g,),
        in_specs=[
            pl.BlockSpec((1, 1, TM4), lambda i: (i, 0, 0)),
            pl.BlockSpec((CODEBOOK_SIZE, CODEBOOK_DIM), lambda i: (0, 0)),
        ],
        out_specs=[pl.BlockSpec((TM4, CODEBOOK_DIM), lambda i: (i, 0))],
        out_shape=[jax.ShapeDtypeStruct((m, CODEBOOK_DIM), jnp.float32)],
    )(idx_flat.reshape(g, 1, TM4), codebook)[0]


N_SC = 4096


def _encode_chunk(xc, W1, b1r, W2, b2r, codebook):
    nc, Dx = xc.shape
    sq = pl.pallas_call(
        _scores_body,
        grid=(nc // TM1,),
        in_specs=[
            pl.BlockSpec((TM1, Dx), lambda i: (i, 0)),
            pl.BlockSpec((Dx, HIDDEN), lambda i: (0, 0)),
            pl.BlockSpec((1, HIDDEN), lambda i: (0, 0)),
            pl.BlockSpec((HIDDEN, CODEBOOK_DIM), lambda i: (0, 0)),
            pl.BlockSpec((1, CODEBOOK_DIM), lambda i: (0, 0)),
            pl.BlockSpec((CODEBOOK_SIZE, CODEBOOK_DIM), lambda i: (0, 0)),
        ],
        out_specs=[pl.BlockSpec((TM1, CODEBOOK_SIZE), lambda i: (i, 0))],
        out_shape=[jax.ShapeDtypeStruct((nc, CODEBOOK_SIZE), jnp.float32)],
    )(xc, W1, b1r, W2, b2r, codebook)[0]

    g2 = nc // TM2
    idx_out, loss_out = pl.pallas_call(
        _argmin_body,
        grid=(g2,),
        in_specs=[pl.BlockSpec((TM2, CODEBOOK_SIZE), lambda i: (i, 0))],
        out_specs=[
            pl.BlockSpec((1, 1, TM2), lambda i: (i, 0, 0)),
            pl.BlockSpec((1, 1), lambda i: (0, 0)),
        ],
        out_shape=[
            jax.ShapeDtypeStruct((g2, 1, TM2), jnp.int32),
            jax.ShapeDtypeStruct((1, 1), jnp.float32),
        ],
    )(sq)
    return idx_out.reshape(nc), loss_out[0, 0]


@jax.jit
def kernel(x, W1, b1, W2, b2, codebook):
    Bx, Tx, Dx = x.shape
    n = Bx * Tx
    xf = x.reshape(n, Dx)
    b1r = b1.reshape(1, HIDDEN)
    b2r = b2.reshape(1, CODEBOOK_DIM)

    idx_flat, loss_sum = _encode_chunk(xf, W1, b1r, W2, b2r, codebook)

    q_sc = _make_gather(N_SC)(codebook, jax.lax.slice(idx_flat, (0,), (N_SC,)))
    q_tc = _tc_gather(jax.lax.slice(idx_flat, (N_SC,), (n,)), codebook)
    quantized = jnp.concatenate([q_sc, q_tc]).reshape(Bx, Tx, CODEBOOK_DIM)

    token_indices = idx_flat.reshape(Bx, Tx)
    denom = jnp.float32(n * CODEBOOK_DIM)
    commitment_loss = loss_sum / denom
    codebook_loss = commitment_loss
    vq_loss = commitment_loss + 0.25 * codebook_loss
    return (token_indices, quantized, commitment_loss, codebook_loss,
            vq_loss)


# speedup vs baseline: 1.0822x; 1.0422x over previous
import functools

import jax
import jax.numpy as jnp
from jax import lax
from jax.experimental import pallas as pl
from jax.experimental.pallas import tpu as pltpu
from jax.experimental.pallas import tpu_sc as plsc

_DEFAULT = jax.lax.Precision.DEFAULT

FEATURE_DIM = 1024
HIDDEN = 512
CODEBOOK_DIM = 512
CODEBOOK_SIZE = 256

TM1 = 512
TM2 = 2048


def _scores_body(x_ref, w1_ref, b1_ref, w2_ref, b2_ref, cb_ref, sq_ref):
    h = jnp.maximum(
        jnp.dot(x_ref[...], w1_ref[...], preferred_element_type=jnp.float32,
                precision=_DEFAULT) + b1_ref[...], 0.0)
    e = jnp.dot(h, w2_ref[...], preferred_element_type=jnp.float32,
                precision=_DEFAULT) + b2_ref[...]
    cb = cb_ref[...]
    scores = jax.lax.dot_general(
        e, cb, (((1,), (1,)), ((), ())),
        preferred_element_type=jnp.float32, precision=_DEFAULT)
    enorm = jnp.sum(e * e, axis=1, keepdims=True)
    cnorm = jnp.sum(cb * cb, axis=1)[None, :]
    sq_ref[...] = enorm + cnorm - 2.0 * scores


def _argmin_body(sq_ref, idx_ref, loss_ref):
    sq = sq_ref[...]
    dist = jnp.sqrt(jnp.maximum(sq, 0.0))
    iota = jax.lax.broadcasted_iota(jnp.int32, (TM2, CODEBOOK_SIZE), 1)
    dmin = jnp.min(dist, axis=1, keepdims=True)
    idx = jnp.min(jnp.where(dist == dmin, iota, CODEBOOK_SIZE),
                  axis=1).astype(jnp.int32)
    idx_ref[0, 0, :] = idx
    onehot = (idx[:, None] == iota).astype(jnp.float32)
    sel = jnp.maximum(jnp.sum(onehot * sq, axis=1), 0.0)
    part = jnp.sum(sel, keepdims=True)[None, :]

    @pl.when(pl.program_id(0) == 0)
    def _():
        loss_ref[...] = jnp.zeros_like(part)

    loss_ref[...] += part


def _make_gather(n):
    info = plsc.get_sparse_core_info()
    nw = info.num_cores * info.num_subcores
    b_per_w = n // nw
    chunk = min(64, b_per_w)
    nchunks = b_per_w // chunk
    mesh = plsc.VectorSubcoreMesh(core_axis_name="c", subcore_axis_name="s")

    @functools.partial(
        pl.kernel, mesh=mesh,
        out_type=jax.ShapeDtypeStruct((n, CODEBOOK_DIM), jnp.float32),
        scratch_types=[
            pltpu.VMEM((b_per_w,), jnp.int32),
            pltpu.VMEM((2, chunk, CODEBOOK_DIM), jnp.float32),
            pltpu.SemaphoreType.DMA,
            pltpu.SemaphoreType.DMA,
        ],
    )
    def gather(table_hbm, idx_hbm, out_hbm, idx_v, rows_v, sem0, sem1):
        wid = lax.axis_index("s") * info.num_cores + lax.axis_index("c")
        base = wid * b_per_w
        pltpu.sync_copy(idx_hbm.at[pl.ds(base, b_per_w)], idx_v)
        sems = (sem0, sem1)

        def fire(c, slot):
            pltpu.async_copy(
                table_hbm.at[idx_v.at[pl.ds(c * chunk, chunk)]],
                rows_v.at[slot], sems[slot])

        fire(0, 0)
        for c in range(nchunks):
            slot = c % 2
            if c + 1 < nchunks:
                fire(c + 1, 1 - slot)
            pltpu.make_async_copy(
                table_hbm.at[idx_v.at[pl.ds(c * chunk, chunk)]],
                rows_v.at[slot], sems[slot]).wait()
            pltpu.sync_copy(rows_v.at[slot],
                            out_hbm.at[pl.ds(base + c * chunk, chunk)])

    return gather


TM4 = 1024


def _tc_gather_body(idx_ref, cb_ref, q_ref):
    idx = idx_ref[0, 0, :]
    onehot = (idx[:, None] == jax.lax.broadcasted_iota(
        jnp.int32, (TM4, CODEBOOK_SIZE), 1)).astype(jnp.float32)
    cb = cb_ref[...]
    cb_hi = cb.astype(jnp.bfloat16).astype(jnp.float32)
    cb_lo = cb - cb_hi
    q_ref[...] = (
        jnp.dot(onehot, cb_hi, preferred_element_type=jnp.float32,
                precision=_DEFAULT)
        + jnp.dot(onehot, cb_lo, preferred_element_type=jnp.float32,
                  precision=_DEFAULT))


def _tc_gather(idx_flat, codebook):
    m = idx_flat.shape[0]
    g = m // TM4
    return pl.pallas_call(
        _tc_gather_body,
        grid=(g,),
        in_specs=[
            pl.BlockSpec((1, 1, TM4), lambda i: (i, 0, 0)),
            pl.BlockSpec((CODEBOOK_SIZE, CODEBOOK_DIM), lambda i: (0, 0)),
        ],
        out_specs=[pl.BlockSpec((TM4, CODEBOOK_DIM), lambda i: (i, 0))],
        out_shape=[jax.ShapeDtypeStruct((m, CODEBOOK_DIM), jnp.float32)],
    )(idx_flat.reshape(g, 1, TM4), codebook)[0]


N_SC = 1024


def _encode_chunk(xc, W1, b1r, W2, b2r, codebook):
    nc, Dx = xc.shape
    sq = pl.pallas_call(
        _scores_body,
        grid=(nc // TM1,),
        in_specs=[
            pl.BlockSpec((TM1, Dx), lambda i: (i, 0)),
            pl.BlockSpec((Dx, HIDDEN), lambda i: (0, 0)),
            pl.BlockSpec((1, HIDDEN), lambda i: (0, 0)),
            pl.BlockSpec((HIDDEN, CODEBOOK_DIM), lambda i: (0, 0)),
            pl.BlockSpec((1, CODEBOOK_DIM), lambda i: (0, 0)),
            pl.BlockSpec((CODEBOOK_SIZE, CODEBOOK_DIM), lambda i: (0, 0)),
        ],
        out_specs=[pl.BlockSpec((TM1, CODEBOOK_SIZE), lambda i: (i, 0))],
        out_shape=[jax.ShapeDtypeStruct((nc, CODEBOOK_SIZE), jnp.float32)],
    )(xc, W1, b1r, W2, b2r, codebook)[0]

    g2 = nc // TM2
    idx_out, loss_out = pl.pallas_call(
        _argmin_body,
        grid=(g2,),
        in_specs=[pl.BlockSpec((TM2, CODEBOOK_SIZE), lambda i: (i, 0))],
        out_specs=[
            pl.BlockSpec((1, 1, TM2), lambda i: (i, 0, 0)),
            pl.BlockSpec((1, 1), lambda i: (0, 0)),
        ],
        out_shape=[
            jax.ShapeDtypeStruct((g2, 1, TM2), jnp.int32),
            jax.ShapeDtypeStruct((1, 1), jnp.float32),
        ],
    )(sq)
    return idx_out.reshape(nc), loss_out[0, 0]


@jax.jit
def kernel(x, W1, b1, W2, b2, codebook):
    Bx, Tx, Dx = x.shape
    n = Bx * Tx
    xf = x.reshape(n, Dx)
    b1r = b1.reshape(1, HIDDEN)
    b2r = b2.reshape(1, CODEBOOK_DIM)

    idx_flat, loss_sum = _encode_chunk(xf, W1, b1r, W2, b2r, codebook)

    q_sc = _make_gather(N_SC)(codebook, jax.lax.slice(idx_flat, (0,), (N_SC,)))
    q_tc = _tc_gather(jax.lax.slice(idx_flat, (N_SC,), (n,)), codebook)
    quantized = jnp.concatenate([q_sc, q_tc]).reshape(Bx, Tx, CODEBOOK_DIM)

    token_indices = idx_flat.reshape(Bx, Tx)
    denom = jnp.float32(n * CODEBOOK_DIM)
    commitment_loss = loss_sum / denom
    codebook_loss = commitment_loss
    vq_loss = commitment_loss + 0.25 * codebook_loss
    return (token_indices, quantized, commitment_loss, codebook_loss,
            vq_loss)


# speedup vs baseline: 1.2803x; 1.1830x over previous
import functools

import jax
import jax.numpy as jnp
from jax import lax
from jax.experimental import pallas as pl
from jax.experimental.pallas import tpu as pltpu
from jax.experimental.pallas import tpu_sc as plsc

_DEFAULT = jax.lax.Precision.DEFAULT

FEATURE_DIM = 1024
HIDDEN = 512
CODEBOOK_DIM = 512
CODEBOOK_SIZE = 256

TM1 = 512
TM2 = 2048


def _scores_body(x_ref, w1_ref, b1_ref, w2_ref, b2_ref, cb_ref, sq_ref):
    h = jnp.maximum(
        jnp.dot(x_ref[...], w1_ref[...], preferred_element_type=jnp.float32,
                precision=_DEFAULT) + b1_ref[...], 0.0)
    e = jnp.dot(h, w2_ref[...], preferred_element_type=jnp.float32,
                precision=_DEFAULT) + b2_ref[...]
    cb = cb_ref[...]
    scores = jax.lax.dot_general(
        e, cb, (((1,), (1,)), ((), ())),
        preferred_element_type=jnp.float32, precision=_DEFAULT)
    enorm = jnp.sum(e * e, axis=1, keepdims=True)
    cnorm = jnp.sum(cb * cb, axis=1)[None, :]
    sq_ref[...] = enorm + cnorm - 2.0 * scores


def _argmin_body(sq_ref, idx_ref, loss_ref):
    sq = sq_ref[...]
    dist = jnp.sqrt(jnp.maximum(sq, 0.0))
    iota = jax.lax.broadcasted_iota(jnp.int32, (TM2, CODEBOOK_SIZE), 1)
    dmin = jnp.min(dist, axis=1, keepdims=True)
    idx = jnp.min(jnp.where(dist == dmin, iota, CODEBOOK_SIZE),
                  axis=1).astype(jnp.int32)
    idx_ref[0, 0, :] = idx
    onehot = (idx[:, None] == iota).astype(jnp.float32)
    sel = jnp.maximum(jnp.sum(onehot * sq, axis=1), 0.0)
    part = jnp.sum(sel, keepdims=True)[None, :]

    @pl.when(pl.program_id(0) == 0)
    def _():
        loss_ref[...] = jnp.zeros_like(part)

    loss_ref[...] += part


def _make_gather(n_total, n_sc):
    info = plsc.get_sparse_core_info()
    nw = info.num_cores * info.num_subcores
    b_per_w = n_sc // nw
    chunk = min(64, b_per_w)
    nchunks = b_per_w // chunk
    mesh = plsc.VectorSubcoreMesh(core_axis_name="c", subcore_axis_name="s")

    @functools.partial(
        pl.kernel, mesh=mesh,
        out_type=jax.ShapeDtypeStruct((n_total, CODEBOOK_DIM), jnp.float32),
        scratch_types=[
            pltpu.VMEM((b_per_w,), jnp.int32),
            pltpu.VMEM((2, chunk, CODEBOOK_DIM), jnp.float32),
            pltpu.SemaphoreType.DMA,
            pltpu.SemaphoreType.DMA,
        ],
    )
    def gather(table_hbm, idx_hbm, out_hbm, idx_v, rows_v, sem0, sem1):
        wid = lax.axis_index("s") * info.num_cores + lax.axis_index("c")
        base = wid * b_per_w
        pltpu.sync_copy(idx_hbm.at[pl.ds(base, b_per_w)], idx_v)
        sems = (sem0, sem1)

        def fire(c, slot):
            pltpu.async_copy(
                table_hbm.at[idx_v.at[pl.ds(c * chunk, chunk)]],
                rows_v.at[slot], sems[slot])

        fire(0, 0)
        for c in range(nchunks):
            slot = c % 2
            if c + 1 < nchunks:
                fire(c + 1, 1 - slot)
            pltpu.make_async_copy(
                table_hbm.at[idx_v.at[pl.ds(c * chunk, chunk)]],
                rows_v.at[slot], sems[slot]).wait()
            pltpu.sync_copy(rows_v.at[slot],
                            out_hbm.at[pl.ds(base + c * chunk, chunk)])

    return gather


TM4 = 1024


def _tc_gather_body(idx_ref, cb_ref, qin_ref, q_ref):
    idx = idx_ref[0, 0, :]
    onehot = (idx[:, None] == jax.lax.broadcasted_iota(
        jnp.int32, (TM4, CODEBOOK_SIZE), 1)).astype(jnp.float32)
    cb = cb_ref[...]
    cb_hi = cb.astype(jnp.bfloat16).astype(jnp.float32)
    cb_lo = cb - cb_hi
    q_ref[...] = (
        jnp.dot(onehot, cb_hi, preferred_element_type=jnp.float32,
                precision=_DEFAULT)
        + jnp.dot(onehot, cb_lo, preferred_element_type=jnp.float32,
                  precision=_DEFAULT))


def _tc_gather(idx_flat, codebook, q_head):
    n = idx_flat.shape[0]
    off = N_SC // TM4
    g = n // TM4 - off
    return pl.pallas_call(
        _tc_gather_body,
        grid=(g,),
        in_specs=[
            pl.BlockSpec((1, 1, TM4), lambda i, off=off: (i + off, 0, 0)),
            pl.BlockSpec((CODEBOOK_SIZE, CODEBOOK_DIM), lambda i: (0, 0)),
            pl.BlockSpec(memory_space=pltpu.MemorySpace.HBM),
        ],
        out_specs=[pl.BlockSpec((TM4, CODEBOOK_DIM),
                                lambda i, off=off: (i + off, 0))],
        out_shape=[jax.ShapeDtypeStruct((n, CODEBOOK_DIM), jnp.float32)],
        input_output_aliases={2: 0},
    )(idx_flat.reshape(n // TM4, 1, TM4), codebook, q_head)[0]


N_SC = 1024


def _encode_chunk(xc, W1, b1r, W2, b2r, codebook):
    nc, Dx = xc.shape
    sq = pl.pallas_call(
        _scores_body,
        grid=(nc // TM1,),
        in_specs=[
            pl.BlockSpec((TM1, Dx), lambda i: (i, 0)),
            pl.BlockSpec((Dx, HIDDEN), lambda i: (0, 0)),
            pl.BlockSpec((1, HIDDEN), lambda i: (0, 0)),
            pl.BlockSpec((HIDDEN, CODEBOOK_DIM), lambda i: (0, 0)),
            pl.BlockSpec((1, CODEBOOK_DIM), lambda i: (0, 0)),
            pl.BlockSpec((CODEBOOK_SIZE, CODEBOOK_DIM), lambda i: (0, 0)),
        ],
        out_specs=[pl.BlockSpec((TM1, CODEBOOK_SIZE), lambda i: (i, 0))],
        out_shape=[jax.ShapeDtypeStruct((nc, CODEBOOK_SIZE), jnp.float32)],
    )(xc, W1, b1r, W2, b2r, codebook)[0]

    g2 = nc // TM2
    idx_out, loss_out = pl.pallas_call(
        _argmin_body,
        grid=(g2,),
        in_specs=[pl.BlockSpec((TM2, CODEBOOK_SIZE), lambda i: (i, 0))],
        out_specs=[
            pl.BlockSpec((1, 1, TM2), lambda i: (i, 0, 0)),
            pl.BlockSpec((1, 1), lambda i: (0, 0)),
        ],
        out_shape=[
            jax.ShapeDtypeStruct((g2, 1, TM2), jnp.int32),
            jax.ShapeDtypeStruct((1, 1), jnp.float32),
        ],
    )(sq)
    return idx_out.reshape(nc), loss_out[0, 0]


@jax.jit
def kernel(x, W1, b1, W2, b2, codebook):
    Bx, Tx, Dx = x.shape
    n = Bx * Tx
    xf = x.reshape(n, Dx)
    b1r = b1.reshape(1, HIDDEN)
    b2r = b2.reshape(1, CODEBOOK_DIM)

    idx_flat, loss_sum = _encode_chunk(xf, W1, b1r, W2, b2r, codebook)

    q_head = _make_gather(n, N_SC)(
        codebook, jax.lax.slice(idx_flat, (0,), (N_SC,)))
    quantized = _tc_gather(idx_flat, codebook, q_head).reshape(
        Bx, Tx, CODEBOOK_DIM)

    token_indices = idx_flat.reshape(Bx, Tx)
    denom = jnp.float32(n * CODEBOOK_DIM)
    commitment_loss = loss_sum / denom
    codebook_loss = commitment_loss
    vq_loss = commitment_loss + 0.25 * codebook_loss
    return (token_indices, quantized, commitment_loss, codebook_loss,
            vq_loss)
